# Initial kernel scaffold; baseline (speedup 1.0000x reference)
import jax
import jax.numpy as jnp
from jax.experimental import pallas as pl


def kernel(x, edge_index, W1, b1, W2, b2):
    raise NotImplementedError("write your pallas kernel here")


# speedup vs baseline: 11.7199x; 11.7199x over previous
import functools

import jax
import jax.numpy as jnp
from jax import lax
from jax.experimental import pallas as pl
from jax.experimental.pallas import tpu as pltpu
from jax.experimental.pallas import tpu_sc as plsc

N_NODES = 10000
D = 128
NC = 2
NS = 16
NW = NC * NS
CHUNK = 128
N_PAD = 10240
ROWS_PER_TILE = N_PAD // NS
DEG_W = 16

_MESH = plsc.VectorSubcoreMesh(core_axis_name="c", subcore_axis_name="s")


def _worker_id():
  return lax.axis_index("c") * NS + lax.axis_index("s")


def _deg_body(epw, dst_hbm, deg_hbm, idx_v, buf_v, acc_sh, sem):
  c = lax.axis_index("c")
  s = lax.axis_index("s")
  wid = _worker_id()

  @pl.loop(0, CHUNK)
  def _(j):
    buf_v[j, :] = jnp.zeros((DEG_W,), jnp.float32)

  @pl.loop(0, ROWS_PER_TILE // CHUNK)
  def _(k):
    pltpu.sync_copy(buf_v, acc_sh.at[pl.ds(s * ROWS_PER_TILE + k * CHUNK, CHUNK)])

  @pl.loop(0, CHUNK)
  def _(j):
    buf_v[j, :] = jnp.ones((DEG_W,), jnp.float32)

  plsc.subcore_barrier()

  @pl.loop(0, epw // CHUNK)
  def _(i):
    pltpu.sync_copy(dst_hbm.at[pl.ds(wid * epw + i * CHUNK, CHUNK)], idx_v)
    pltpu.sync_copy(buf_v, acc_sh.at[idx_v], add=True)

  plsc.subcore_barrier()

  @pl.loop(0, ROWS_PER_TILE // CHUNK)
  def _(k):
    r0 = s * ROWS_PER_TILE + k * CHUNK
    pltpu.sync_copy(acc_sh.at[pl.ds(r0, CHUNK)], buf_v)
    pltpu.sync_copy(buf_v, deg_hbm.at[c].at[pl.ds(r0, CHUNK)])


def _sc_degree(dst_pad):
  epw = dst_pad.shape[0] // NW
  k = functools.partial(
      pl.kernel,
      out_type=jax.ShapeDtypeStruct((NC, N_PAD, DEG_W), jnp.float32),
      mesh=_MESH,
      scratch_types=[
          pltpu.VMEM((CHUNK,), jnp.int32),
          pltpu.VMEM((CHUNK, DEG_W), jnp.float32),
          pltpu.VMEM_SHARED((N_PAD, DEG_W), jnp.float32),
          pltpu.SemaphoreType.DMA,
      ],
  )(functools.partial(_deg_body, epw))
  return k(dst_pad)


def _scat_body(epw, y_hbm, src_hbm, dst_hbm, out_hbm, srcv, dstv, rows_v,
               acc_sh, sem):
  c = lax.axis_index("c")
  s = lax.axis_index("s")
  wid = _worker_id()

  @pl.loop(0, CHUNK)
  def _(j):
    for k in range(D // 16):
      rows_v[j, pl.ds(k * 16, 16)] = jnp.zeros((16,), jnp.float32)

  @pl.loop(0, ROWS_PER_TILE // CHUNK)
  def _(k):
    pltpu.sync_copy(rows_v, acc_sh.at[pl.ds(s * ROWS_PER_TILE + k * CHUNK, CHUNK)])

  plsc.subcore_barrier()

  @pl.loop(0, epw // CHUNK)
  def _(i):
    base = wid * epw + i * CHUNK
    pltpu.sync_copy(src_hbm.at[pl.ds(base, CHUNK)], srcv)
    pltpu.sync_copy(dst_hbm.at[pl.ds(base, CHUNK)], dstv)
    pltpu.async_copy(y_hbm.at[srcv], rows_v, sem).wait()
    pltpu.sync_copy(rows_v, acc_sh.at[dstv], add=True)

  plsc.subcore_barrier()

  @pl.loop(0, ROWS_PER_TILE // CHUNK)
  def _(k):
    r0 = s * ROWS_PER_TILE + k * CHUNK
    pltpu.sync_copy(acc_sh.at[pl.ds(r0, CHUNK)], rows_v)
    pltpu.sync_copy(rows_v, out_hbm.at[c].at[pl.ds(r0, CHUNK)])


def _sc_scatter(y, src_pad, dst_pad):
  epw = src_pad.shape[0] // NW
  k = functools.partial(
      pl.kernel,
      out_type=jax.ShapeDtypeStruct((NC, N_PAD, D), jnp.float32),
      mesh=_MESH,
      scratch_types=[
          pltpu.VMEM((CHUNK,), jnp.int32),
          pltpu.VMEM((CHUNK,), jnp.int32),
          pltpu.VMEM((CHUNK, D), jnp.float32),
          pltpu.VMEM_SHARED((N_PAD, D), jnp.float32),
          pltpu.SemaphoreType.DMA,
      ],
  )(functools.partial(_scat_body, epw))
  return k(y, src_pad, dst_pad)


_BLK = 2000


def _mm_body(x_ref, w_ref, o_ref):
  o_ref[...] = jnp.dot(x_ref[...], w_ref[...],
                       preferred_element_type=jnp.float32)


def _tc_matmul(x, w):
  return pl.pallas_call(
      _mm_body,
      grid=(N_NODES // _BLK,),
      in_specs=[pl.BlockSpec((_BLK, D), lambda i: (i, 0)),
                pl.BlockSpec((D, D), lambda i: (0, 0))],
      out_specs=pl.BlockSpec((_BLK, D), lambda i: (i, 0)),
      out_shape=jax.ShapeDtypeStruct((N_NODES, D), jnp.float32),
  )(x, w)


def _scale1_body(xw_ref, deg_ref, y_ref, dis_ref):
  d = deg_ref[0] + deg_ref[1] + 1.0
  dis = lax.rsqrt(d)
  dis_ref[...] = dis
  y_ref[...] = dis[:, :1] * xw_ref[...]


def _tc_scale1(xw, deg):
  return pl.pallas_call(
      _scale1_body,
      grid=(N_NODES // _BLK,),
      in_specs=[pl.BlockSpec((_BLK, D), lambda i: (i, 0)),
                pl.BlockSpec((NC, _BLK, DEG_W), lambda i: (0, i, 0))],
      out_specs=[pl.BlockSpec((_BLK, D), lambda i: (i, 0)),
                 pl.BlockSpec((_BLK, DEG_W), lambda i: (i, 0))],
      out_shape=[jax.ShapeDtypeStruct((N_NODES, D), jnp.float32),
                 jax.ShapeDtypeStruct((N_NODES, DEG_W), jnp.float32)],
  )(xw, deg)


def _mid_body(acc_ref, y1_ref, dis_ref, b1_ref, w2_ref, y2_ref):
  tot = acc_ref[0] + acc_ref[1] + y1_ref[...]
  dis = dis_ref[:, :1]
  h = jnp.maximum(dis * tot + b1_ref[...], 0.0)
  y2_ref[...] = dis * jnp.dot(h, w2_ref[...],
                              preferred_element_type=jnp.float32)


def _tc_mid(acc1, y1, dis, b1, w2):
  return pl.pallas_call(
      _mid_body,
      grid=(N_NODES // _BLK,),
      in_specs=[pl.BlockSpec((NC, _BLK, D), lambda i: (0, i, 0)),
                pl.BlockSpec((_BLK, D), lambda i: (i, 0)),
                pl.BlockSpec((_BLK, DEG_W), lambda i: (i, 0)),
                pl.BlockSpec((1, D), lambda i: (0, 0)),
                pl.BlockSpec((D, D), lambda i: (0, 0))],
      out_specs=pl.BlockSpec((_BLK, D), lambda i: (i, 0)),
      out_shape=jax.ShapeDtypeStruct((N_NODES, D), jnp.float32),
  )(acc1, y1, dis, b1, w2)


def _final_body(acc_ref, y2_ref, dis_ref, b2_ref, z_ref):
  tot = acc_ref[0] + acc_ref[1] + y2_ref[...]
  z_ref[...] = dis_ref[:, :1] * tot + b2_ref[...]


def _tc_final(acc2, y2, dis, b2):
  return pl.pallas_call(
      _final_body,
      grid=(N_NODES // _BLK,),
      in_specs=[pl.BlockSpec((NC, _BLK, D), lambda i: (0, i, 0)),
                pl.BlockSpec((_BLK, D), lambda i: (i, 0)),
                pl.BlockSpec((_BLK, DEG_W), lambda i: (i, 0)),
                pl.BlockSpec((1, D), lambda i: (0, 0))],
      out_specs=pl.BlockSpec((_BLK, D), lambda i: (i, 0)),
      out_shape=jax.ShapeDtypeStruct((N_NODES, D), jnp.float32),
  )(acc2, y2, dis, b2)


def kernel(x, edge_index, W1, b1, W2, b2):
  e = edge_index.shape[1]
  epw = -(-e // (NW * CHUNK)) * CHUNK
  e_pad = epw * NW
  src = edge_index[0].astype(jnp.int32)
  dst = edge_index[1].astype(jnp.int32)
  src_pad = jnp.concatenate([src, jnp.zeros((e_pad - e,), jnp.int32)])
  dst_pad = jnp.concatenate(
      [dst, jnp.full((e_pad - e,), N_NODES, jnp.int32)])
  b1r = b1.reshape(1, D)
  b2r = b2.reshape(1, D)

  deg = _sc_degree(dst_pad)
  xw1 = _tc_matmul(x, W1)
  y1, dis = _tc_scale1(xw1, deg[:, :N_NODES])
  acc1 = _sc_scatter(y1, src_pad, dst_pad)
  y2 = _tc_mid(acc1[:, :N_NODES], y1, dis, b1r, W2)
  acc2 = _sc_scatter(y2, src_pad, dst_pad)
  z = _tc_final(acc2[:, :N_NODES], y2, dis, b2r)
  return z
